# baseline (speedup 1.0000x reference)
import functools

import jax
import jax.numpy as jnp
from jax import lax
from jax.experimental import pallas as pl
from jax.experimental.pallas import tpu as pltpu
from jax.experimental.pallas import tpu_sc as plsc

B_ROWS = 4096
SEQ = 200
D = 64
SCALE = 8.0
VOCAB_HALF = 500000

B_TOTAL = B_ROWS * SEQ
NC = 2
NS = 16
NW = NC * NS
B_PER_W = B_TOTAL // NW

IDX_MINOR = 128
SUP = 1024
SUP_ROWS = SUP // IDX_MINOR
CHUNK = 128
N_SUB = SUP // CHUNK
N_SUPS = B_PER_W // SUP
RBLK = 16

_mesh = plsc.VectorSubcoreMesh(core_axis_name="c", subcore_axis_name="s")


@functools.partial(
    pl.kernel,
    mesh=_mesh,
    out_type=jax.ShapeDtypeStruct((B_TOTAL // 2, 2 * D), jnp.float32),
    scratch_types=[
        pltpu.VMEM((SUP_ROWS, IDX_MINOR), jnp.int32),
        pltpu.VMEM((SUP_ROWS, IDX_MINOR), jnp.int32),
        pltpu.VMEM((2, CHUNK, 2 * D), jnp.float32),
        pltpu.VMEM((2, CHUNK // 2, 2 * D), jnp.float32),
        pltpu.SemaphoreType.DMA((2,)),
        pltpu.SemaphoreType.DMA((2,)),
    ],
    compiler_params=pltpu.CompilerParams(needs_layout_passes=False),
)
def _embed(idx_hbm, sup_hbm, tab_hbm, out_hbm, idx_v, sup_v, rows_v, out_v,
           gsem, osem):
    wid = lax.axis_index("s") * NC + lax.axis_index("c")
    base = wid * B_PER_W
    _iota16 = lax.broadcasted_iota(jnp.int32, (RBLK,), 0)

    def gather(k, b):
        return pltpu.make_async_copy(
            tab_hbm.at[sup_v.at[k]], rows_v.at[b], gsem.at[b])

    def outcopy(g, c, b):
        off = pl.multiple_of((base + (g * N_SUB + c) * CHUNK) // 2, 8)
        return pltpu.make_async_copy(
            out_v.at[b], out_hbm.at[pl.ds(off, CHUNK // 2)], osem.at[b])

    def select(c, b):
        @plsc.parallel_loop(0, CHUNK // RBLK, 1, unroll=2)
        def select_rows(rb):
            r0 = rb * RBLK
            row16 = r0 + _iota16
            idx16 = idx_v[c, pl.ds(r0, RBLK)]
            col0 = (idx16 & 1) * D
            dr16 = lax.shift_right_logical(row16, 1)
            dc0 = (row16 & 1) * D
            for j in range(D):
                v = plsc.load_gather(rows_v.at[b], [row16, col0 + j])
                plsc.store_scatter(out_v.at[b], [dr16, dc0 + j],
                                   v * SCALE)

    def sup_body(g, carry):
        slab = pl.multiple_of((base + g * SUP) // IDX_MINOR, 8)
        pltpu.sync_copy(idx_hbm.at[pl.ds(slab, SUP_ROWS)], idx_v)
        pltpu.sync_copy(sup_hbm.at[pl.ds(slab, SUP_ROWS)], sup_v)
        gather(0, 0).start()
        gather(1, 1).start()

        @pl.when(g > 0)
        def _():
            outcopy(g - 1, N_SUB - 2, 0).wait()
            outcopy(g - 1, N_SUB - 1, 1).wait()

        def pair_body(p, pc):
            for s in range(2):
                c = p * 2 + s
                b = s
                gather(c, b).wait()

                @pl.when(p > 0)
                def _():
                    outcopy(g, c - 2, b).wait()

                select(c, b)
                @pl.when(c + 2 < N_SUB)
                def _():
                    gather(c + 2, b).start()
                outcopy(g, c, b).start()
            return pc

        lax.fori_loop(0, N_SUB // 2, pair_body, 0)
        return carry

    lax.fori_loop(0, N_SUPS, sup_body, 0)
    outcopy(N_SUPS - 1, N_SUB - 2, 0).wait()
    outcopy(N_SUPS - 1, N_SUB - 1, 1).wait()


def kernel(x, lut):
    idx = x.reshape(B_TOTAL // IDX_MINOR, IDX_MINOR).astype(jnp.int32)
    sup = lax.shift_right_logical(idx, 1)
    tabw = lut.reshape(VOCAB_HALF, 2 * D)
    out2 = _embed(idx, sup, tabw)
    return out2.reshape(B_ROWS, SEQ, D)


# speedup vs baseline: 1.3721x; 1.3721x over previous
import functools

import jax
import jax.numpy as jnp
from jax import lax
from jax.experimental import pallas as pl
from jax.experimental.pallas import tpu as pltpu
from jax.experimental.pallas import tpu_sc as plsc

B_ROWS = 4096
SEQ = 200
D = 64
SCALE = 8.0
VOCAB_HALF = 500000

B_TOTAL = B_ROWS * SEQ
NC = 2
NS = 16
NW = NC * NS
B_PER_W = B_TOTAL // NW

IDX_MINOR = 128
SUP = 1024
SUP_ROWS = SUP // IDX_MINOR
CHUNK = 256
N_SUB = SUP // CHUNK
N_GATH = CHUNK // IDX_MINOR
N_SUPS = B_PER_W // SUP
RBLK = 16

_mesh = plsc.VectorSubcoreMesh(core_axis_name="c", subcore_axis_name="s")


@functools.partial(
    pl.kernel,
    mesh=_mesh,
    out_type=jax.ShapeDtypeStruct((B_TOTAL // 2, 2 * D), jnp.float32),
    scratch_types=[
        pltpu.VMEM((SUP_ROWS, IDX_MINOR), jnp.int32),
        pltpu.VMEM((SUP_ROWS, IDX_MINOR), jnp.int32),
        pltpu.VMEM((CHUNK, 2 * D), jnp.float32),
        pltpu.VMEM((CHUNK // 2, 2 * D), jnp.float32),
        pltpu.SemaphoreType.DMA,
    ],
    compiler_params=pltpu.CompilerParams(needs_layout_passes=False,
                                         disable_bounds_checks=True),
)
def _embed(idx_hbm, sup_hbm, tab_hbm, out_hbm, idx_v, sup_v, rows_v,
           out_v, sem):
    wid = lax.axis_index("s") * NC + lax.axis_index("c")
    base = wid * B_PER_W

    def sup_body(g, carry):
        sup_off = base + g * SUP
        slab = pl.multiple_of(sup_off // IDX_MINOR, 8)
        pltpu.sync_copy(idx_hbm.at[pl.ds(slab, SUP_ROWS)], idx_v)
        pltpu.sync_copy(sup_hbm.at[pl.ds(slab, SUP_ROWS)], sup_v)

        for c in range(N_SUB):
            for k in range(N_GATH):
                pltpu.async_copy(
                    tab_hbm.at[sup_v.at[c * N_GATH + k]],
                    rows_v.at[pl.ds(k * IDX_MINOR, IDX_MINOR)],
                    sem,
                )
            for k in range(N_GATH):
                pltpu.make_async_copy(
                    tab_hbm.at[sup_v.at[c * N_GATH + k]],
                    rows_v.at[pl.ds(k * IDX_MINOR, IDX_MINOR)],
                    sem,
                ).wait()

            def select_rows(rb, cr):
                r0 = rb * RBLK
                sr = c * CHUNK + r0
                idx16 = idx_v[sr // IDX_MINOR, pl.ds(sr % IDX_MINOR, RBLK)]
                par16 = idx16 & 1
                dr0 = rb * (RBLK // 2)
                for l in range(RBLK):
                    r = r0 + l
                    lane = lax.broadcasted_iota(jnp.int32, (RBLK,), 0) * 0 + l
                    odd = jnp.take(par16, lane) == 1
                    dr = dr0 + l // 2
                    db = (l % 2) * D
                    for j in range(D // RBLK):
                        a = rows_v[r, pl.ds(j * RBLK, RBLK)]
                        b = rows_v[r, pl.ds(D + j * RBLK, RBLK)]
                        out_v[dr, pl.ds(db + j * RBLK, RBLK)] = (
                            jnp.where(odd, b, a) * SCALE)
                return cr

            lax.fori_loop(0, CHUNK // RBLK, select_rows, 0)
            out_off = pl.multiple_of((sup_off + c * CHUNK) // 2, 8)
            pltpu.sync_copy(out_v, out_hbm.at[pl.ds(out_off, CHUNK // 2)])
        return carry

    lax.fori_loop(0, N_SUPS, sup_body, 0)


def kernel(x, lut):
    idx = x.reshape(B_TOTAL // IDX_MINOR, IDX_MINOR).astype(jnp.int32)
    sup = lax.shift_right_logical(idx, 1)
    tabw = lut.reshape(VOCAB_HALF, 2 * D)
    out2 = _embed(idx, sup, tabw)
    return out2.reshape(B_ROWS, SEQ, D)


# speedup vs baseline: 2.0740x; 1.5115x over previous
import functools

import jax
import jax.numpy as jnp
from jax import lax
from jax.experimental import pallas as pl
from jax.experimental.pallas import tpu as pltpu
from jax.experimental.pallas import tpu_sc as plsc

B_ROWS = 4096
SEQ = 200
D = 64
SCALE = 8.0

B_TOTAL = B_ROWS * SEQ
NC = 2
NS = 16
NW = NC * NS
B_PER_W = B_TOTAL // NW

IDX_MINOR = 128
CHUNK = 1024
N_GATH = CHUNK // IDX_MINOR
N_CHUNKS = B_PER_W // CHUNK

_mesh = plsc.VectorSubcoreMesh(core_axis_name="c", subcore_axis_name="s")


@functools.partial(
    pl.kernel,
    mesh=_mesh,
    out_type=jax.ShapeDtypeStruct((B_TOTAL, D), jnp.float32),
    scratch_types=[
        pltpu.VMEM((N_GATH, IDX_MINOR), jnp.int32),
        pltpu.VMEM((CHUNK, D), jnp.float32),
        pltpu.SemaphoreType.DMA,
    ],
    compiler_params=pltpu.CompilerParams(use_tc_tiling_on_sc=False),
)
def _embed(idx_hbm, tab_hbm, out_hbm, idx_v, rows_v, sem):
    wid = lax.axis_index("s") * NC + lax.axis_index("c")
    base = wid * B_PER_W

    def chunk_body(g, carry):
        off = base + g * CHUNK
        idx_off = pl.multiple_of(off // IDX_MINOR, 8)
        pltpu.sync_copy(idx_hbm.at[pl.ds(idx_off, N_GATH)], idx_v)
        for k in range(N_GATH):
            pltpu.async_copy(
                tab_hbm.at[idx_v.at[k]],
                rows_v.at[pl.ds(k * IDX_MINOR, IDX_MINOR)],
                sem,
            )
        for k in range(N_GATH):
            pltpu.make_async_copy(
                tab_hbm.at[idx_v.at[k]],
                rows_v.at[pl.ds(k * IDX_MINOR, IDX_MINOR)],
                sem,
            ).wait()
        def scale_row(r, c):
            for j in range(D // 16):
                sl = pl.ds(j * 16, 16)
                rows_v[r, sl] = rows_v[r, sl] * SCALE
            return c

        lax.fori_loop(0, CHUNK, scale_row, 0, unroll=2)
        pltpu.sync_copy(rows_v, out_hbm.at[pl.ds(off, CHUNK)])
        return carry

    lax.fori_loop(0, N_CHUNKS, chunk_body, 0)


def kernel(x, lut):
    idx2 = x.reshape(B_TOTAL // IDX_MINOR, IDX_MINOR).astype(jnp.int32)
    out = _embed(idx2, lut)
    return out.reshape(B_ROWS, SEQ, D)
